# Initial kernel scaffold; baseline (speedup 1.0000x reference)
import jax
import jax.numpy as jnp
from jax.experimental import pallas as pl


def kernel(x, edge_index, W1, b1, W2, b2, Wf1, bf1, Wf2, bf2):
    raise NotImplementedError("write your pallas kernel here")


# speedup vs baseline: 6.6243x; 6.6243x over previous
import functools

import jax
import jax.numpy as jnp
from jax import lax
from jax.experimental import pallas as pl
from jax.experimental.pallas import tpu as pltpu
from jax.experimental.pallas import tpu_sc as plsc

N = 10000
NP = 10240
E = 160000
EP = 163840
K = 128
NCHUNK = EP // K
NC, NS = 2, 16
ROWS_PER_TILE = NP // NS


def _fill_f32(ref, rows, cols, val):
    def row_body(i, _):
        def col_body(k, _):
            ref[i, pl.ds(k * 16, 16)] = jnp.full((16,), val, jnp.float32)
            return 0
        return lax.fori_loop(0, cols // 16, col_body, 0)
    lax.fori_loop(0, rows, row_body, 0)


def _deg_body(dst2d, degp, ones_v, didx, zbuf, acc):
    c = lax.axis_index("c")
    s = lax.axis_index("s")
    w = c * NS + s
    _fill_f32(ones_v, K, 16, 1.0)
    _fill_f32(zbuf, 64, 16, 0.0)
    def zloop(k, _):
        pltpu.sync_copy(zbuf, acc.at[pl.ds(s * ROWS_PER_TILE + k * 64, 64)])
        return 0
    lax.fori_loop(0, ROWS_PER_TILE // 64, zloop, 0)
    nchunks = NCHUNK // (NC * NS)
    pltpu.sync_copy(dst2d.at[pl.ds(w * nchunks, nchunks)], didx)
    plsc.subcore_barrier()
    def body(j, _):
        pltpu.sync_copy(ones_v, acc.at[didx.at[j]], add=True)
        return 0
    lax.fori_loop(0, nchunks, body, 0)
    plsc.subcore_barrier()
    pltpu.sync_copy(
        acc.at[pl.ds(s * ROWS_PER_TILE, ROWS_PER_TILE)],
        degp.at[pl.ds(c * NP + s * ROWS_PER_TILE, ROWS_PER_TILE)],
    )


_deg_kernel = functools.partial(
    pl.kernel,
    out_type=jax.ShapeDtypeStruct((NC * NP, 16), jnp.float32),
    mesh=plsc.VectorSubcoreMesh(core_axis_name="c", subcore_axis_name="s"),
    scratch_types=[
        pltpu.VMEM((K, 16), jnp.float32),
        pltpu.VMEM((NCHUNK // (NC * NS), K), jnp.int32),
        pltpu.VMEM((64, 16), jnp.float32),
        pltpu.VMEM_SHARED((NP, 16), jnp.float32),
    ],
)(_deg_body)


def _agg1_body(g_st, src_pair, dst2d, out_st, sidx, didx, rows_v, zbuf, acc, sem):
    c = lax.axis_index("c")
    s = lax.axis_index("s")
    nchunks = NCHUNK // NS
    _fill_f32(zbuf, 8, 128, 0.0)
    def zloop(k, _):
        pltpu.sync_copy(zbuf, acc.at[pl.ds(s * ROWS_PER_TILE + k * 8, 8)])
        return 0
    lax.fori_loop(0, ROWS_PER_TILE // 8, zloop, 0)
    pltpu.sync_copy(src_pair.at[pl.ds(c * NCHUNK + s * nchunks, nchunks)], sidx)
    pltpu.sync_copy(dst2d.at[pl.ds(s * nchunks, nchunks)], didx)
    plsc.subcore_barrier()
    def body(j, _):
        pltpu.async_copy(g_st.at[sidx.at[j]], rows_v, sem).wait()
        pltpu.sync_copy(rows_v, acc.at[didx.at[j]], add=True)
        return 0
    lax.fori_loop(0, nchunks, body, 0)
    plsc.subcore_barrier()
    pltpu.sync_copy(
        acc.at[pl.ds(s * ROWS_PER_TILE, ROWS_PER_TILE)],
        out_st.at[pl.ds(c * NP + s * ROWS_PER_TILE, ROWS_PER_TILE)],
    )


_agg1_kernel = functools.partial(
    pl.kernel,
    out_type=jax.ShapeDtypeStruct((NC * NP, 128), jnp.float32),
    mesh=plsc.VectorSubcoreMesh(core_axis_name="c", subcore_axis_name="s"),
    scratch_types=[
        pltpu.VMEM((NCHUNK // NS, K), jnp.int32),
        pltpu.VMEM((NCHUNK // NS, K), jnp.int32),
        pltpu.VMEM((K, 128), jnp.float32),
        pltpu.VMEM((8, 128), jnp.float32),
        pltpu.VMEM_SHARED((NP, 128), jnp.float32),
        pltpu.SemaphoreType.DMA,
    ],
)(_agg1_body)


def _agg2_body(g2, src2d, dst2d, out_st, sidx, didx, rows_v, zbuf, acc, sem):
    c = lax.axis_index("c")
    s = lax.axis_index("s")
    w = c * NS + s
    nchunks = NCHUNK // (NC * NS)
    _fill_f32(zbuf, 8, 128, 0.0)
    def zloop(k, _):
        pltpu.sync_copy(zbuf, acc.at[pl.ds(s * ROWS_PER_TILE + k * 8, 8)])
        return 0
    lax.fori_loop(0, ROWS_PER_TILE // 8, zloop, 0)
    pltpu.sync_copy(src2d.at[pl.ds(w * nchunks, nchunks)], sidx)
    pltpu.sync_copy(dst2d.at[pl.ds(w * nchunks, nchunks)], didx)
    plsc.subcore_barrier()
    def body(j, _):
        pltpu.async_copy(g2.at[sidx.at[j]], rows_v, sem).wait()
        pltpu.sync_copy(rows_v, acc.at[didx.at[j]], add=True)
        return 0
    lax.fori_loop(0, nchunks, body, 0)
    plsc.subcore_barrier()
    pltpu.sync_copy(
        acc.at[pl.ds(s * ROWS_PER_TILE, ROWS_PER_TILE)],
        out_st.at[pl.ds(c * NP + s * ROWS_PER_TILE, ROWS_PER_TILE)],
    )


_agg2_kernel = functools.partial(
    pl.kernel,
    out_type=jax.ShapeDtypeStruct((NC * NP, 128), jnp.float32),
    mesh=plsc.VectorSubcoreMesh(core_axis_name="c", subcore_axis_name="s"),
    scratch_types=[
        pltpu.VMEM((NCHUNK // (NC * NS), K), jnp.int32),
        pltpu.VMEM((NCHUNK // (NC * NS), K), jnp.int32),
        pltpu.VMEM((K, 128), jnp.float32),
        pltpu.VMEM((8, 128), jnp.float32),
        pltpu.VMEM_SHARED((NP, 128), jnp.float32),
        pltpu.SemaphoreType.DMA,
    ],
)(_agg2_body)


_RB = 256
_NBLK = NP // _RB


def _dinv_block(d0, d1):
    return lax.rsqrt(d0[:, 0:1] + d1[:, 0:1] + 1.0)


def _rowmask(i):
    r = i * _RB + lax.broadcasted_iota(jnp.int32, (_RB, 1), 0)
    return r < N


def _tc_b_body(x_ref, w_ref, d0_ref, d1_ref, o_ref):
    i = pl.program_id(1)
    dinv = _dinv_block(d0_ref[...], d1_ref[...])
    m = jnp.dot(x_ref[...], w_ref[...], preferred_element_type=jnp.float32)
    o_ref[...] = jnp.where(_rowmask(i), m * dinv, 0.0)


def _tc_b(xp, W1, d0, d1):
    return pl.pallas_call(
        _tc_b_body,
        grid=(2, _NBLK),
        in_specs=[
            pl.BlockSpec((_RB, 256), lambda h, i: (i, 0)),
            pl.BlockSpec((256, 128), lambda h, i: (0, h)),
            pl.BlockSpec((_RB, 16), lambda h, i: (i, 0)),
            pl.BlockSpec((_RB, 16), lambda h, i: (i, 0)),
        ],
        out_specs=pl.BlockSpec((_RB, 128), lambda h, i: (h * _NBLK + i, 0)),
        out_shape=jax.ShapeDtypeStruct((NC * NP, 128), jnp.float32),
    )(xp, W1, d0, d1)


def _tc_d_body(alo_ref, ahi_ref, glo_ref, ghi_ref, d0_ref, d1_ref, b1_ref,
               w2_ref, o_ref):
    i = pl.program_id(0)
    dinv = _dinv_block(d0_ref[...], d1_ref[...])
    b1 = b1_ref[...]
    hl = jnp.maximum((alo_ref[...] + glo_ref[...]) * dinv + b1[:, 0:128], 0.0)
    hh = jnp.maximum((ahi_ref[...] + ghi_ref[...]) * dinv + b1[:, 128:256], 0.0)
    w2 = w2_ref[...]
    m2 = (jnp.dot(hl, w2[0:128, :], preferred_element_type=jnp.float32)
          + jnp.dot(hh, w2[128:256, :], preferred_element_type=jnp.float32))
    o_ref[...] = jnp.where(_rowmask(i), m2 * dinv, 0.0)


def _tc_d(agg_st, g_st, d0, d1, b1r, W2):
    return pl.pallas_call(
        _tc_d_body,
        grid=(_NBLK,),
        in_specs=[
            pl.BlockSpec((_RB, 128), lambda i: (i, 0)),
            pl.BlockSpec((_RB, 128), lambda i: (_NBLK + i, 0)),
            pl.BlockSpec((_RB, 128), lambda i: (i, 0)),
            pl.BlockSpec((_RB, 128), lambda i: (_NBLK + i, 0)),
            pl.BlockSpec((_RB, 16), lambda i: (i, 0)),
            pl.BlockSpec((_RB, 16), lambda i: (i, 0)),
            pl.BlockSpec((1, 256), lambda i: (0, 0)),
            pl.BlockSpec((256, 128), lambda i: (0, 0)),
        ],
        out_specs=pl.BlockSpec((_RB, 128), lambda i: (i, 0)),
        out_shape=jax.ShapeDtypeStruct((NP, 128), jnp.float32),
    )(agg_st, agg_st, g_st, g_st, d0, d1, b1r, W2)


def _tc_f_body(plo_ref, phi_ref, g2_ref, d0_ref, d1_ref, b2_ref, wfc_ref,
               bfc_ref, o_ref):
    dinv = _dinv_block(d0_ref[...], d1_ref[...])
    h2 = jnp.maximum(
        (plo_ref[...] + phi_ref[...] + g2_ref[...]) * dinv + b2_ref[...], 0.0)
    o_ref[...] = (jnp.dot(h2, wfc_ref[...], preferred_element_type=jnp.float32)
                  + bfc_ref[...])


def _tc_f(agg2_st, g2, d0, d1, b2r, Wfc, bfcr):
    return pl.pallas_call(
        _tc_f_body,
        grid=(_NBLK,),
        in_specs=[
            pl.BlockSpec((_RB, 128), lambda i: (i, 0)),
            pl.BlockSpec((_RB, 128), lambda i: (_NBLK + i, 0)),
            pl.BlockSpec((_RB, 128), lambda i: (i, 0)),
            pl.BlockSpec((_RB, 16), lambda i: (i, 0)),
            pl.BlockSpec((_RB, 16), lambda i: (i, 0)),
            pl.BlockSpec((1, 128), lambda i: (0, 0)),
            pl.BlockSpec((128, 256), lambda i: (0, 0)),
            pl.BlockSpec((1, 256), lambda i: (0, 0)),
        ],
        out_specs=pl.BlockSpec((_RB, 256), lambda i: (i, 0)),
        out_shape=jax.ShapeDtypeStruct((NP, 256), jnp.float32),
    )(agg2_st, agg2_st, g2, d0, d1, b2r, Wfc, bfcr)


def _tc_w_body(wf1_ref, wf2_ref, bf1_ref, bf2_ref, wfc_ref, bfc_ref):
    wf2 = wf2_ref[...]
    wfc_ref[...] = jnp.dot(wf1_ref[...], wf2, preferred_element_type=jnp.float32)
    bfc_ref[...] = (jnp.dot(bf1_ref[...], wf2, preferred_element_type=jnp.float32)
                    + bf2_ref[...])


def _tc_w(Wf1, Wf2, bf1r, bf2r):
    return pl.pallas_call(
        _tc_w_body,
        out_shape=(jax.ShapeDtypeStruct((128, 256), jnp.float32),
                   jax.ShapeDtypeStruct((1, 256), jnp.float32)),
    )(Wf1, Wf2, bf1r, bf2r)


def kernel(x, edge_index, W1, b1, W2, b2, Wf1, bf1, Wf2, bf2):
    src = edge_index[0]
    dst = edge_index[1]
    pad = jnp.full((EP - E,), N, dtype=jnp.int32)
    src2d = jnp.concatenate([src, pad]).reshape(NCHUNK, K)
    dst2d = jnp.concatenate([dst, pad]).reshape(NCHUNK, K)
    src_pair = jnp.concatenate([src2d, src2d + NP], axis=0)
    xp = jnp.pad(x, ((0, NP - N), (0, 0)))
    b1r = b1.reshape(1, 256)
    b2r = b2.reshape(1, 128)
    bf1r = bf1.reshape(1, 256)
    bf2r = bf2.reshape(1, 256)

    degp = _deg_kernel(dst2d)
    d0 = degp[:NP]
    d1 = degp[NP:]
    Wfc, bfcr = _tc_w(Wf1, Wf2, bf1r, bf2r)

    g1_st = _tc_b(xp, W1, d0, d1)
    agg1_st = _agg1_kernel(g1_st, src_pair, dst2d)
    g2 = _tc_d(agg1_st, g1_st, d0, d1, b1r, W2)
    agg2_st = _agg2_kernel(g2, src2d, dst2d)
    outp = _tc_f(agg2_st, g2, d0, d1, b2r, Wfc, bfcr)
    return outp[:N]


# speedup vs baseline: 7.5078x; 1.1334x over previous
import functools

import jax
import jax.numpy as jnp
from jax import lax
from jax.experimental import pallas as pl
from jax.experimental.pallas import tpu as pltpu
from jax.experimental.pallas import tpu_sc as plsc

N = 10000
NP = 10240
E = 160000
EP = 163840
K = 128
NCHUNK = EP // K
NC, NS = 2, 16
ROWS_PER_TILE = NP // NS
_PH = 2


def _fill_f32(ref, rows, cols, val):
    def row_body(i, _):
        def col_body(k, _):
            ref[i, pl.ds(k * 16, 16)] = jnp.full((16,), val, jnp.float32)
            return 0
        return lax.fori_loop(0, cols // 16, col_body, 0)
    lax.fori_loop(0, rows, row_body, 0)


def _deg_body(dst2d, degp, ones_v, didx, zbuf, acc):
    c = lax.axis_index("c")
    s = lax.axis_index("s")
    w = c * NS + s
    _fill_f32(ones_v, K, 16, 1.0)
    _fill_f32(zbuf, 64, 16, 0.0)
    def zloop(k, _):
        pltpu.sync_copy(zbuf, acc.at[pl.ds(s * ROWS_PER_TILE + k * 64, 64)])
        return 0
    lax.fori_loop(0, ROWS_PER_TILE // 64, zloop, 0)
    nchunks = NCHUNK // (NC * NS)
    pltpu.sync_copy(dst2d.at[pl.ds(w * nchunks, nchunks)], didx)
    plsc.subcore_barrier()
    def body(j, _):
        pltpu.sync_copy(ones_v, acc.at[didx.at[j]], add=True)
        return 0
    lax.fori_loop(0, nchunks, body, 0)
    plsc.subcore_barrier()
    pltpu.sync_copy(
        acc.at[pl.ds(s * ROWS_PER_TILE, ROWS_PER_TILE)],
        degp.at[pl.ds(c * NP + s * ROWS_PER_TILE, ROWS_PER_TILE)],
    )


_deg_kernel = functools.partial(
    pl.kernel,
    out_type=jax.ShapeDtypeStruct((NC * NP, 16), jnp.float32),
    mesh=plsc.VectorSubcoreMesh(core_axis_name="c", subcore_axis_name="s"),
    scratch_types=[
        pltpu.VMEM((K, 16), jnp.float32),
        pltpu.VMEM((NCHUNK // (NC * NS), K), jnp.int32),
        pltpu.VMEM((64, 16), jnp.float32),
        pltpu.VMEM_SHARED((NP, 16), jnp.float32),
    ],
)(_deg_body)


def _gather_scatter_loop(table, sidx, didx, rows_a, rows_b, sem_a, sem_b, acc,
                         nchunks):
    half = nchunks // 2
    pltpu.async_copy(table.at[sidx.at[0]], rows_a, sem_a)

    def body(p, _):
        ja = 2 * p
        jb = 2 * p + 1
        pltpu.async_copy(table.at[sidx.at[jb]], rows_b, sem_b)
        pltpu.make_async_copy(table.at[sidx.at[ja]], rows_a, sem_a).wait()
        pltpu.sync_copy(rows_a, acc.at[didx.at[ja]], add=True)

        @pl.when(p < half - 1)
        def _():
            pltpu.async_copy(table.at[sidx.at[ja + 2]], rows_a, sem_a)

        pltpu.make_async_copy(table.at[sidx.at[jb]], rows_b, sem_b).wait()
        pltpu.sync_copy(rows_b, acc.at[didx.at[jb]], add=True)
        return 0

    lax.fori_loop(0, half, body, 0)


def _agg1_body(g_st, src_pair, dst2d, out_st, sidx, didx, rows_a, rows_b, zbuf,
               acc, sem_a, sem_b):
    c = lax.axis_index("c")
    s = lax.axis_index("s")
    nchunks = NCHUNK // NS
    nph = nchunks // _PH
    _fill_f32(zbuf, 8, 128, 0.0)
    def zloop(k, _):
        pltpu.sync_copy(zbuf, acc.at[pl.ds(s * ROWS_PER_TILE + k * 8, 8)])
        return 0
    lax.fori_loop(0, ROWS_PER_TILE // 8, zloop, 0)
    plsc.subcore_barrier()
    def phase(ph, _):
        pltpu.sync_copy(
            src_pair.at[pl.ds(c * NCHUNK + s * nchunks + ph * nph, nph)], sidx)
        pltpu.sync_copy(dst2d.at[pl.ds(s * nchunks + ph * nph, nph)], didx)
        _gather_scatter_loop(g_st, sidx, didx, rows_a, rows_b, sem_a, sem_b,
                             acc, nph)
        return 0
    lax.fori_loop(0, _PH, phase, 0)
    plsc.subcore_barrier()
    pltpu.sync_copy(
        acc.at[pl.ds(s * ROWS_PER_TILE, ROWS_PER_TILE)],
        out_st.at[pl.ds(c * NP + s * ROWS_PER_TILE, ROWS_PER_TILE)],
    )


_agg1_kernel = functools.partial(
    pl.kernel,
    out_type=jax.ShapeDtypeStruct((NC * NP, 128), jnp.float32),
    mesh=plsc.VectorSubcoreMesh(core_axis_name="c", subcore_axis_name="s"),
    scratch_types=[
        pltpu.VMEM((NCHUNK // NS // _PH, K), jnp.int32),
        pltpu.VMEM((NCHUNK // NS // _PH, K), jnp.int32),
        pltpu.VMEM((K, 128), jnp.float32),
        pltpu.VMEM((K, 128), jnp.float32),
        pltpu.VMEM((8, 128), jnp.float32),
        pltpu.VMEM_SHARED((NP, 128), jnp.float32),
        pltpu.SemaphoreType.DMA,
        pltpu.SemaphoreType.DMA,
    ],
)(_agg1_body)


def _agg2_body(g2, src2d, dst2d, out_st, sidx, didx, rows_a, rows_b, zbuf, acc,
               sem_a, sem_b):
    c = lax.axis_index("c")
    s = lax.axis_index("s")
    w = c * NS + s
    nchunks = NCHUNK // (NC * NS)
    _fill_f32(zbuf, 8, 128, 0.0)
    def zloop(k, _):
        pltpu.sync_copy(zbuf, acc.at[pl.ds(s * ROWS_PER_TILE + k * 8, 8)])
        return 0
    lax.fori_loop(0, ROWS_PER_TILE // 8, zloop, 0)
    pltpu.sync_copy(src2d.at[pl.ds(w * nchunks, nchunks)], sidx)
    pltpu.sync_copy(dst2d.at[pl.ds(w * nchunks, nchunks)], didx)
    plsc.subcore_barrier()
    _gather_scatter_loop(g2, sidx, didx, rows_a, rows_b, sem_a, sem_b, acc,
                         nchunks)
    plsc.subcore_barrier()
    pltpu.sync_copy(
        acc.at[pl.ds(s * ROWS_PER_TILE, ROWS_PER_TILE)],
        out_st.at[pl.ds(c * NP + s * ROWS_PER_TILE, ROWS_PER_TILE)],
    )


_agg2_kernel = functools.partial(
    pl.kernel,
    out_type=jax.ShapeDtypeStruct((NC * NP, 128), jnp.float32),
    mesh=plsc.VectorSubcoreMesh(core_axis_name="c", subcore_axis_name="s"),
    scratch_types=[
        pltpu.VMEM((NCHUNK // (NC * NS), K), jnp.int32),
        pltpu.VMEM((NCHUNK // (NC * NS), K), jnp.int32),
        pltpu.VMEM((K, 128), jnp.float32),
        pltpu.VMEM((K, 128), jnp.float32),
        pltpu.VMEM((8, 128), jnp.float32),
        pltpu.VMEM_SHARED((NP, 128), jnp.float32),
        pltpu.SemaphoreType.DMA,
        pltpu.SemaphoreType.DMA,
    ],
)(_agg2_body)


_RB = 256
_NBLK = NP // _RB


def _dinv_block(d0, d1):
    return lax.rsqrt(d0[:, 0:1] + d1[:, 0:1] + 1.0)


def _rowmask(i):
    r = i * _RB + lax.broadcasted_iota(jnp.int32, (_RB, 1), 0)
    return r < N


def _tc_b_body(x_ref, w_ref, d0_ref, d1_ref, o_ref):
    i = pl.program_id(1)
    dinv = _dinv_block(d0_ref[...], d1_ref[...])
    m = jnp.dot(x_ref[...], w_ref[...], preferred_element_type=jnp.float32)
    o_ref[...] = jnp.where(_rowmask(i), m * dinv, 0.0)


def _tc_b(xp, W1, d0, d1):
    return pl.pallas_call(
        _tc_b_body,
        grid=(2, _NBLK),
        in_specs=[
            pl.BlockSpec((_RB, 256), lambda h, i: (i, 0)),
            pl.BlockSpec((256, 128), lambda h, i: (0, h)),
            pl.BlockSpec((_RB, 16), lambda h, i: (i, 0)),
            pl.BlockSpec((_RB, 16), lambda h, i: (i, 0)),
        ],
        out_specs=pl.BlockSpec((_RB, 128), lambda h, i: (h * _NBLK + i, 0)),
        out_shape=jax.ShapeDtypeStruct((NC * NP, 128), jnp.float32),
    )(xp, W1, d0, d1)


def _tc_d_body(alo_ref, ahi_ref, glo_ref, ghi_ref, d0_ref, d1_ref, b1_ref,
               w2_ref, o_ref):
    i = pl.program_id(0)
    dinv = _dinv_block(d0_ref[...], d1_ref[...])
    b1 = b1_ref[...]
    hl = jnp.maximum((alo_ref[...] + glo_ref[...]) * dinv + b1[:, 0:128], 0.0)
    hh = jnp.maximum((ahi_ref[...] + ghi_ref[...]) * dinv + b1[:, 128:256], 0.0)
    w2 = w2_ref[...]
    m2 = (jnp.dot(hl, w2[0:128, :], preferred_element_type=jnp.float32)
          + jnp.dot(hh, w2[128:256, :], preferred_element_type=jnp.float32))
    o_ref[...] = jnp.where(_rowmask(i), m2 * dinv, 0.0)


def _tc_d(agg_st, g_st, d0, d1, b1r, W2):
    return pl.pallas_call(
        _tc_d_body,
        grid=(_NBLK,),
        in_specs=[
            pl.BlockSpec((_RB, 128), lambda i: (i, 0)),
            pl.BlockSpec((_RB, 128), lambda i: (_NBLK + i, 0)),
            pl.BlockSpec((_RB, 128), lambda i: (i, 0)),
            pl.BlockSpec((_RB, 128), lambda i: (_NBLK + i, 0)),
            pl.BlockSpec((_RB, 16), lambda i: (i, 0)),
            pl.BlockSpec((_RB, 16), lambda i: (i, 0)),
            pl.BlockSpec((1, 256), lambda i: (0, 0)),
            pl.BlockSpec((256, 128), lambda i: (0, 0)),
        ],
        out_specs=pl.BlockSpec((_RB, 128), lambda i: (i, 0)),
        out_shape=jax.ShapeDtypeStruct((NP, 128), jnp.float32),
    )(agg_st, agg_st, g_st, g_st, d0, d1, b1r, W2)


def _tc_f_body(plo_ref, phi_ref, g2_ref, d0_ref, d1_ref, b2_ref, wfc_ref,
               bfc_ref, o_ref):
    dinv = _dinv_block(d0_ref[...], d1_ref[...])
    h2 = jnp.maximum(
        (plo_ref[...] + phi_ref[...] + g2_ref[...]) * dinv + b2_ref[...], 0.0)
    o_ref[...] = (jnp.dot(h2, wfc_ref[...], preferred_element_type=jnp.float32)
                  + bfc_ref[...])


def _tc_f(agg2_st, g2, d0, d1, b2r, Wfc, bfcr):
    return pl.pallas_call(
        _tc_f_body,
        grid=(_NBLK,),
        in_specs=[
            pl.BlockSpec((_RB, 128), lambda i: (i, 0)),
            pl.BlockSpec((_RB, 128), lambda i: (_NBLK + i, 0)),
            pl.BlockSpec((_RB, 128), lambda i: (i, 0)),
            pl.BlockSpec((_RB, 16), lambda i: (i, 0)),
            pl.BlockSpec((_RB, 16), lambda i: (i, 0)),
            pl.BlockSpec((1, 128), lambda i: (0, 0)),
            pl.BlockSpec((128, 256), lambda i: (0, 0)),
            pl.BlockSpec((1, 256), lambda i: (0, 0)),
        ],
        out_specs=pl.BlockSpec((_RB, 256), lambda i: (i, 0)),
        out_shape=jax.ShapeDtypeStruct((NP, 256), jnp.float32),
    )(agg2_st, agg2_st, g2, d0, d1, b2r, Wfc, bfcr)


def _tc_w_body(wf1_ref, wf2_ref, bf1_ref, bf2_ref, wfc_ref, bfc_ref):
    wf2 = wf2_ref[...]
    wfc_ref[...] = jnp.dot(wf1_ref[...], wf2, preferred_element_type=jnp.float32)
    bfc_ref[...] = (jnp.dot(bf1_ref[...], wf2, preferred_element_type=jnp.float32)
                    + bf2_ref[...])


def _tc_w(Wf1, Wf2, bf1r, bf2r):
    return pl.pallas_call(
        _tc_w_body,
        out_shape=(jax.ShapeDtypeStruct((128, 256), jnp.float32),
                   jax.ShapeDtypeStruct((1, 256), jnp.float32)),
    )(Wf1, Wf2, bf1r, bf2r)


def kernel(x, edge_index, W1, b1, W2, b2, Wf1, bf1, Wf2, bf2):
    src = edge_index[0]
    dst = edge_index[1]
    pad = jnp.full((EP - E,), N, dtype=jnp.int32)
    src2d = jnp.concatenate([src, pad]).reshape(NCHUNK, K)
    dst2d = jnp.concatenate([dst, pad]).reshape(NCHUNK, K)
    src_pair = jnp.concatenate([src2d, src2d + NP], axis=0)
    xp = jnp.pad(x, ((0, NP - N), (0, 0)))
    b1r = b1.reshape(1, 256)
    b2r = b2.reshape(1, 128)
    bf1r = bf1.reshape(1, 256)
    bf2r = bf2.reshape(1, 256)

    degp = _deg_kernel(dst2d)
    d0 = degp[:NP]
    d1 = degp[NP:]
    Wfc, bfcr = _tc_w(Wf1, Wf2, bf1r, bf2r)

    g1_st = _tc_b(xp, W1, d0, d1)
    agg1_st = _agg1_kernel(g1_st, src_pair, dst2d)
    g2 = _tc_d(agg1_st, g1_st, d0, d1, b1r, W2)
    agg2_st = _agg2_kernel(g2, src2d, dst2d)
    outp = _tc_f(agg2_st, g2, d0, d1, b2r, Wfc, bfcr)
    return outp[:N]
